# Initial kernel scaffold; baseline (speedup 1.0000x reference)
import jax
import jax.numpy as jnp
from jax.experimental import pallas as pl


def kernel(x, local_cell_indices_nh, W):
    raise NotImplementedError("write your pallas kernel here")


# speedup vs baseline: 25.9600x; 25.9600x over previous
import functools

import jax
import jax.numpy as jnp
from jax import lax
from jax.experimental import pallas as pl
from jax.experimental.pallas import tpu as pltpu
from jax.experimental.pallas import tpu_sc as plsc

B, N, D, NH = 8, 10000, 128, 16
TOTAL = B * N
LANES = 16
NW = 32
CHUNK_NODES = 40
CHUNK_IDX = CHUNK_NODES * NH
IDX_PER_XFER = 128
XFERS = CHUNK_IDX // IDX_PER_XFER
N_CHUNKS = TOTAL // CHUNK_NODES
CHUNKS_PER_W = -(-N_CHUNKS // NW)

MM_BLK = 640


def _mm_body(x_ref, w_ref, o_ref):
    o_ref[...] = jnp.dot(x_ref[...], w_ref[...] * (1.0 / NH),
                         preferred_element_type=jnp.float32)


def _matmul(x2, w):
    return pl.pallas_call(
        _mm_body,
        grid=(TOTAL // MM_BLK,),
        in_specs=[pl.BlockSpec((MM_BLK, D), lambda i: (i, 0)),
                  pl.BlockSpec((D, D), lambda i: (0, 0))],
        out_specs=pl.BlockSpec((MM_BLK, D), lambda i: (i, 0)),
        out_shape=jax.ShapeDtypeStruct((TOTAL, D), jnp.float32),
    )(x2, w)


def _sc_gather_sum(y2, gidx):
    mesh = plsc.VectorSubcoreMesh(core_axis_name="c", subcore_axis_name="s")

    @functools.partial(
        pl.kernel,
        mesh=mesh,
        out_type=jax.ShapeDtypeStruct((TOTAL, D), jnp.float32),
        scratch_types=[
            pltpu.VMEM((CHUNK_IDX,), jnp.int32),
            pltpu.VMEM((CHUNK_IDX, D), jnp.float32),
            pltpu.VMEM((CHUNK_NODES, D), jnp.float32),
            pltpu.SemaphoreType.DMA,
        ],
    )
    def k(y_hbm, gidx_hbm, out_hbm, idx_v, rows_v, out_v, sem):
        wid = lax.axis_index("s") * 2 + lax.axis_index("c")

        def chunk_body(kk, _):
            cid = wid + kk * NW

            @pl.when(cid < N_CHUNKS)
            def _():
                node0 = cid * CHUNK_NODES
                off_vec = jnp.full((LANES,), (node0 // N) * N, dtype=jnp.int32)
                pltpu.sync_copy(gidx_hbm.at[pl.ds(cid * CHUNK_IDX, CHUNK_IDX)],
                                idx_v)
                for i in range(CHUNK_IDX // LANES):
                    sl = pl.ds(i * LANES, LANES)
                    idx_v[sl] = idx_v[sl] + off_vec
                cps = [pltpu.async_copy(
                           y_hbm.at[idx_v.at[pl.ds(j * IDX_PER_XFER,
                                                   IDX_PER_XFER)]],
                           rows_v.at[pl.ds(j * IDX_PER_XFER, IDX_PER_XFER)],
                           sem)
                       for j in range(XFERS)]
                for cp in cps:
                    cp.wait()

                def node_body(m, _):
                    r0 = m * NH
                    for l in range(D // LANES):
                        sl = pl.ds(l * LANES, LANES)
                        acc = rows_v[r0, sl]
                        for r in range(1, NH):
                            acc = acc + rows_v[r0 + r, sl]
                        out_v[m, sl] = acc
                    return 0

                lax.fori_loop(0, CHUNK_NODES, node_body, 0)
                pltpu.sync_copy(out_v,
                                out_hbm.at[pl.ds(node0, CHUNK_NODES)])

            return 0

        lax.fori_loop(0, CHUNKS_PER_W, chunk_body, 0)

    return k(y2, gidx)


def kernel(x, local_cell_indices_nh, W):
    x2 = x.reshape(TOTAL, D)
    y2 = _matmul(x2, W)
    gidx = local_cell_indices_nh.astype(jnp.int32).reshape(TOTAL * NH)
    out2 = _sc_gather_sum(y2, gidx)
    return out2.reshape(B, N, D)


# speedup vs baseline: 33.0264x; 1.2722x over previous
import functools

import jax
import jax.numpy as jnp
from jax import lax
from jax.experimental import pallas as pl
from jax.experimental.pallas import tpu as pltpu
from jax.experimental.pallas import tpu_sc as plsc

B, N, D, NH = 8, 10000, 128, 16
TOTAL = B * N
LANES = 16
NW = 32
CHUNK_NODES = 16
CHUNK_IDX = CHUNK_NODES * NH
IDX_PER_XFER = 128
XFERS = CHUNK_IDX // IDX_PER_XFER
N_CHUNKS = TOTAL // CHUNK_NODES
STEPS = -(-N_CHUNKS // NW)
PAIRS = (STEPS + 2) // 2

MM_BLK = 640


def _mm_body(x_ref, w_ref, o_ref):
    o_ref[...] = jnp.dot(x_ref[...], w_ref[...] * (1.0 / NH),
                         preferred_element_type=jnp.float32)


def _matmul(x2, w):
    return pl.pallas_call(
        _mm_body,
        grid=(TOTAL // MM_BLK,),
        in_specs=[pl.BlockSpec((MM_BLK, D), lambda i: (i, 0)),
                  pl.BlockSpec((D, D), lambda i: (0, 0))],
        out_specs=pl.BlockSpec((MM_BLK, D), lambda i: (i, 0)),
        out_shape=jax.ShapeDtypeStruct((TOTAL, D), jnp.float32),
    )(x2, w)


def _sc_gather_sum(y2, gidx):
    mesh = plsc.VectorSubcoreMesh(core_axis_name="c", subcore_axis_name="s")

    @functools.partial(
        pl.kernel,
        mesh=mesh,
        out_type=jax.ShapeDtypeStruct((TOTAL, D), jnp.float32),
        scratch_types=[
            pltpu.VMEM((CHUNK_IDX,), jnp.int32),
            pltpu.VMEM((CHUNK_IDX,), jnp.int32),
            pltpu.VMEM((CHUNK_IDX, D), jnp.float32),
            pltpu.VMEM((CHUNK_IDX, D), jnp.float32),
            pltpu.VMEM((CHUNK_NODES, D), jnp.float32),
            pltpu.SemaphoreType.DMA,
            pltpu.SemaphoreType.DMA,
        ],
    )
    def k(y_hbm, gidx_hbm, out_hbm, idx0, idx1, rows0, rows1, out_v,
          sem0, sem1):
        wid = lax.axis_index("s") * 2 + lax.axis_index("c")
        idx_bufs = (idx0, idx1)
        row_bufs = (rows0, rows1)
        sems = (sem0, sem1)

        def gather_copies(idx_v, rows_v, sem):
            return [pltpu.make_async_copy(
                        y_hbm.at[idx_v.at[pl.ds(j * IDX_PER_XFER,
                                                IDX_PER_XFER)]],
                        rows_v.at[pl.ds(j * IDX_PER_XFER, IDX_PER_XFER)],
                        sem)
                    for j in range(XFERS)]

        def fire(t, p):
            cid = wid + t * NW

            @pl.when(cid < N_CHUNKS)
            def _():
                idx_v = idx_bufs[p]
                node0 = cid * CHUNK_NODES
                off_vec = jnp.full((LANES,), (node0 // N) * N,
                                   dtype=jnp.int32)
                pltpu.sync_copy(
                    gidx_hbm.at[pl.ds(cid * CHUNK_IDX, CHUNK_IDX)], idx_v)
                for i in range(CHUNK_IDX // LANES):
                    sl = pl.ds(i * LANES, LANES)
                    idx_v[sl] = idx_v[sl] + off_vec
                for cp in gather_copies(idx_v, row_bufs[p], sems[p]):
                    cp.start()

        def consume(t, p):
            cid = wid + t * NW

            @pl.when(cid < N_CHUNKS)
            def _():
                rows_v = row_bufs[p]
                for cp in gather_copies(idx_bufs[p], rows_v, sems[p]):
                    cp.wait()

                def node_body(m, _):
                    r0 = m * NH
                    for l in range(D // LANES):
                        sl = pl.ds(l * LANES, LANES)
                        acc = rows_v[r0, sl]
                        for r in range(1, NH):
                            acc = acc + rows_v[r0 + r, sl]
                        out_v[m, sl] = acc
                    return 0

                lax.fori_loop(0, CHUNK_NODES, node_body, 0)
                pltpu.sync_copy(out_v,
                                out_hbm.at[pl.ds(cid * CHUNK_NODES,
                                                 CHUNK_NODES)])

        fire(0, 0)

        def pair_body(kk, _):
            t = 2 * kk
            fire(t + 1, 1)
            consume(t, 0)
            fire(t + 2, 0)
            consume(t + 1, 1)
            return 0

        lax.fori_loop(0, PAIRS, pair_body, 0)

    return k(y2, gidx)


def kernel(x, local_cell_indices_nh, W):
    x2 = x.reshape(TOTAL, D)
    y2 = _matmul(x2, W)
    gidx = local_cell_indices_nh.astype(jnp.int32).reshape(TOTAL * NH)
    out2 = _sc_gather_sum(y2, gidx)
    return out2.reshape(B, N, D)
